# baseline (speedup 1.0000x reference)
import functools

import jax
import jax.numpy as jnp
from jax import lax
from jax.experimental import pallas as pl
from jax.experimental.pallas import tpu as pltpu
from jax.experimental.pallas import tpu_sc as plsc

_EPS = 1e-8
_MARGIN = 0.6
_C = 256
_F = 32
_HW = _F * _F
_IMG = 256
_NW = 32
_CELLS = 64


def _tc_kernel(sq_ref, rk_ref, tm_ref):
    xq = sq_ref[0]
    xr = rk_ref[0]
    qn = jnp.maximum(jnp.sqrt(jnp.sum(xq * xq, axis=0, keepdims=True)), _EPS)
    rn = jnp.maximum(jnp.sqrt(jnp.sum(xr * xr, axis=0, keepdims=True)), _EPS)
    xqn = xq / qn
    xrn = xr / rn
    st = jax.lax.dot_general(
        xrn, xqn, (((0,), (0,)), ((), ())),
        preferred_element_type=jnp.float32,
        precision=jax.lax.Precision.HIGHEST,
    )
    row = jax.lax.broadcasted_iota(jnp.int32, (_HW, _HW), 0)
    m1 = jnp.min(st, axis=0, keepdims=True)
    j1 = jnp.min(jnp.where(st == m1, row, _HW), axis=0, keepdims=True)
    s1 = jnp.where(row == j1, jnp.inf, st)
    m2 = jnp.min(s1, axis=0, keepdims=True)
    j2 = jnp.min(jnp.where(s1 == m2, row, _HW), axis=0, keepdims=True)
    s2 = jnp.where(row == j2, jnp.inf, s1)
    m3 = jnp.min(s2, axis=0, keepdims=True)
    bot3 = m1 + m2 + m3
    tm_ref[0] = (bot3 + _MARGIN) - st


def _sc_kernel(tm_hbm, g_hbm, out_hbm, tm_v, g_v, o_v):
    wid = lax.axis_index("s") * 2 + lax.axis_index("c")
    b = wid // 16
    chunk = wid % 16
    g0 = chunk * _CELLS
    fy0 = chunk * 2
    pltpu.sync_copy(tm_hbm.at[b, pl.ds(g0 * _HW, _CELLS * _HW)], tm_v)
    pltpu.sync_copy(g_hbm.at[b, 4 + fy0 * 8, :], g_v.at[pl.ds(0, _IMG * 2)])
    pltpu.sync_copy(g_hbm.at[b, 12 + fy0 * 8, :], g_v.at[pl.ds(_IMG * 2, _IMG * 2)])
    num = jnp.zeros((16,), jnp.float32)
    den = jnp.zeros((16,), jnp.float32)
    iota = lax.iota(jnp.int32, 16)
    for v in range(4):
        c = iota + (v * 16)
        r = c >> 5
        fx = c & 31
        colx = r * (_IMG * 2) + (fx << 4) + 8
        gx = plsc.load_gather(g_v, [colx])
        gy = plsc.load_gather(g_v, [colx + 1])
        px = gx * 256.0
        py = gy * 256.0
        tx = px.astype(jnp.int32)
        ty = py.astype(jnp.int32)
        xmin = tx - (tx.astype(jnp.float32) > px).astype(jnp.int32)
        ymin = ty - (ty.astype(jnp.float32) > py).astype(jnp.int32)
        valid = (xmin >= 0) & (ymin >= 0) & (xmin <= 255) & (ymin <= 255)
        x0 = xmin >> 3
        x1 = (xmin + 1) >> 3
        y0 = ymin >> 3
        y1 = (ymin + 1) >> 3
        mx0 = (x0 >= 0) & (x0 <= _F)
        mx1 = (x1 != x0) & (x1 >= 0) & (x1 <= _F)
        my0 = (y0 >= 0) & (y0 <= _F)
        my1 = (y1 != y0) & (y1 >= 0) & (y1 <= _F)
        for xs, ys, mj in ((x0, y0, mx0 & my0), (x0, y1, mx0 & my1),
                           (x1, y0, mx1 & my0), (x1, y1, mx1 & my1)):
            ia = jnp.clip(ys, 0, _F - 1) * _F + jnp.clip(xs, 0, _F - 1)
            val = plsc.load_gather(tm_v, [c * _HW + ia])
            m = mj & valid
            num = num + jnp.where(m, jnp.maximum(val, 0.0), 0.0)
            den = den + jnp.where(m, 1.0, 0.0)
    o_v[pl.ds(0, 16)] = num
    o_v[pl.ds(16, 16)] = den
    pltpu.sync_copy(o_v, out_hbm.at[wid])


def kernel(sketch_query_vectors, ref_key_vectors, G):
    B = sketch_query_vectors.shape[0]
    sq = sketch_query_vectors.reshape(B, _C, _HW)
    rk = ref_key_vectors.reshape(B, _C, _HW)
    g2 = G.reshape(B, _IMG, _IMG * 2)
    tm = pl.pallas_call(
        _tc_kernel,
        grid=(B,),
        in_specs=[
            pl.BlockSpec((1, _C, _HW), lambda i: (i, 0, 0)),
            pl.BlockSpec((1, _C, _HW), lambda i: (i, 0, 0)),
        ],
        out_specs=pl.BlockSpec((1, _HW, _HW), lambda i: (i, 0, 0)),
        out_shape=jax.ShapeDtypeStruct((B, _HW, _HW), jnp.float32),
    )(sq, rk)
    sc_fn = functools.partial(
        pl.kernel,
        mesh=plsc.VectorSubcoreMesh(core_axis_name="c", subcore_axis_name="s"),
        compiler_params=pltpu.CompilerParams(needs_layout_passes=False),
        out_type=jax.ShapeDtypeStruct((_NW, 32), jnp.float32),
        scratch_types=[
            pltpu.VMEM((_CELLS * _HW,), jnp.float32),
            pltpu.VMEM((2 * _IMG * 2,), jnp.float32),
            pltpu.VMEM((32,), jnp.float32),
        ],
    )(_sc_kernel)
    parts = sc_fn(tm.reshape(B, _HW * _HW), g2)
    return parts[:, :16].sum() / (1e-6 + parts[:, 16:].sum())
